# Initial kernel scaffold; baseline (speedup 1.0000x reference)
import jax
import jax.numpy as jnp
from jax.experimental import pallas as pl


def kernel(x, edge_index, W1, att_src1, att_dst1, b1, W2, att_src2, att_dst2, b2):
    raise NotImplementedError("write your pallas kernel here")


# speedup vs baseline: 1.0294x; 1.0294x over previous
import jax
import jax.numpy as jnp
from jax.experimental import pallas as pl


def _mm_body(x_ref, w_ref, o_ref):
    o_ref[...] = jnp.dot(x_ref[...], w_ref[...],
                         preferred_element_type=jnp.float32)


def _matmul(x, w):
    return pl.pallas_call(
        _mm_body,
        out_shape=jax.ShapeDtypeStruct((x.shape[0], w.shape[1]), jnp.float32),
    )(x, w)


def _gat_conv(x, edge_index, W, att_src, att_dst, bias, heads, out_ch):
    n = x.shape[0]
    loops = jnp.arange(n, dtype=edge_index.dtype)
    src = jnp.concatenate([edge_index[0], loops])
    dst = jnp.concatenate([edge_index[1], loops])
    h = _matmul(x, W).reshape(n, heads, out_ch)
    a_s = jnp.sum(h * att_src[None, :, :], axis=-1)
    a_d = jnp.sum(h * att_dst[None, :, :], axis=-1)
    e = a_s[src] + a_d[dst]
    e = jax.nn.leaky_relu(e, negative_slope=0.2)
    m = jax.ops.segment_max(e, dst, num_segments=n)
    ex = jnp.exp(e - m[dst])
    denom = jax.ops.segment_sum(ex, dst, num_segments=n)
    alpha = ex / (denom[dst] + 1e-16)
    out = jax.ops.segment_sum(h[src] * alpha[:, :, None], dst, num_segments=n)
    out = out.reshape(n, heads * out_ch)
    return out + bias


def kernel(x, edge_index, W1, att_src1, att_dst1, b1, W2, att_src2,
           att_dst2, b2):
    h = _gat_conv(x, edge_index, W1, att_src1, att_dst1, b1, 8, 8)
    h = _gat_conv(h, edge_index, W2, att_src2, att_dst2, b2, 1, 7)
    return jax.nn.log_softmax(h, axis=1)
